# baseline (speedup 1.0000x reference)
import functools

import numpy as np
import jax
import jax.numpy as jnp
from jax import lax
from jax.experimental import pallas as pl
from jax.experimental.pallas import tpu as pltpu
from jax.experimental.pallas import tpu_sc as plsc

B = 4096
S = 200
D = 64
N = B * S
NW = 32
PER_W = N // NW
CH = 128
NCH = PER_W // CH


def _positional_encoding_np():
    pos = np.arange(S, dtype=np.float32)[:, None]
    idx = np.arange(D, dtype=np.float32)[None, :]
    angle_rates = 1.0 / np.power(
        np.float32(10000.0), 2.0 * np.floor(idx / 2.0) / np.float32(D)
    )
    angle_rads = (pos * angle_rates).astype(np.float32)
    sines = np.sin(angle_rads[:, 0::2])
    cosines = np.cos(angle_rads[:, 1::2])
    pe = np.concatenate([sines[:, :, None], cosines[:, :, None]], axis=-1)
    return pe.reshape(S, D).astype(np.float32)


_PE2 = np.concatenate([_positional_encoding_np()] * 2, axis=0)


@functools.partial(
    pl.kernel,
    out_type=jax.ShapeDtypeStruct((N, D), jnp.float32),
    mesh=plsc.VectorSubcoreMesh(core_axis_name="c", subcore_axis_name="s"),
    compiler_params=pltpu.CompilerParams(use_tc_tiling_on_sc=False),
    scratch_types=[
        pltpu.VMEM((PER_W,), jnp.int32),
        pltpu.VMEM((2 * S, D), jnp.float32),
        pltpu.VMEM((CH, D), jnp.float32),
        pltpu.VMEM((CH, D), jnp.float32),
        pltpu.VMEM((CH, D), jnp.float32),
        pltpu.VMEM((CH, D), jnp.float32),
        pltpu.SemaphoreType.DMA,
        pltpu.SemaphoreType.DMA,
        pltpu.SemaphoreType.DMA,
        pltpu.SemaphoreType.DMA,
    ],
)
def _emb_lookup(table_hbm, x_hbm, pe_hbm, out_hbm,
                idx_v, pe_v, g0, g1, o0, o1, gs0, gs1, os0, os1):
    wid = lax.axis_index("s") * 2 + lax.axis_index("c")
    base = wid * PER_W

    pltpu.sync_copy(x_hbm.at[pl.ds(base, PER_W)], idx_v)
    pltpu.sync_copy(pe_hbm, pe_v)

    def gather_start(c, gbuf, gsem):
        pltpu.make_async_copy(
            table_hbm.at[idx_v.at[pl.ds(c * CH, CH)]], gbuf, gsem
        ).start()

    def gather_wait(gbuf, gsem):
        pltpu.make_async_copy(
            table_hbm.at[idx_v.at[pl.ds(0, CH)]], gbuf, gsem
        ).wait()

    def out_start(c, obuf, osem):
        pltpu.make_async_copy(
            obuf, out_hbm.at[pl.ds(base + c * CH, CH)], osem
        ).start()

    def out_wait(obuf, osem):
        pltpu.make_async_copy(
            obuf, out_hbm.at[pl.ds(base, CH)], osem
        ).wait()

    def compute(c, gbuf, obuf):
        j0 = lax.rem(c * CH, S)

        def body(u, carry):
            j = j0 + u
            for f in range(D // 16):
                sl = pl.ds(f * 16, 16)
                obuf[u, sl] = gbuf[u, sl] * 8.0 + pe_v[j, sl]
            return carry

        lax.fori_loop(0, CH, body, 0)

    def step(c, cc, gbuf, obuf, gsem, osem):
        gather_wait(gbuf, gsem)

        @pl.when(cc > 0)
        def _():
            out_wait(obuf, osem)

        compute(c, gbuf, obuf)
        out_start(c, obuf, osem)

        @pl.when(c + 2 < NCH)
        def _():
            gather_start(c + 2, gbuf, gsem)

    gather_start(0, g0, gs0)
    gather_start(1, g1, gs1)

    def loop_body(cc, carry):
        step(cc * 2, cc, g0, o0, gs0, os0)
        step(cc * 2 + 1, cc, g1, o1, gs1, os1)
        return carry

    lax.fori_loop(0, NCH // 2, loop_body, 0)
    out_wait(o0, os0)
    out_wait(o1, os1)


def kernel(x, table):
    x_flat = x.reshape(-1).astype(jnp.int32)
    pe2 = jnp.asarray(_PE2)
    out = _emb_lookup(table, x_flat, pe2)
    return out.reshape(B, S, D)
